# baseline (speedup 1.0000x reference)
import functools

import jax
import jax.numpy as jnp
from jax import lax
from jax.experimental import pallas as pl
from jax.experimental.pallas import tpu as pltpu
from jax.experimental.pallas import tpu_sc as plsc

_VOCAB = 1000000
_EMBED = 32
_BATCH = 4096
_GROUP = 128 // _EMBED
_NGROUPS = _VOCAB // _GROUP

_NUM_CORES = 2
_NUM_SUBCORES = 16
_NUM_WORKERS = _NUM_CORES * _NUM_SUBCORES
_PER_WORKER = _BATCH // _NUM_WORKERS
_LANES = 16


def _make_gather():
  mesh = plsc.VectorSubcoreMesh(
      core_axis_name="c", subcore_axis_name="s",
      num_cores=_NUM_CORES, num_subcores=_NUM_SUBCORES)

  @functools.partial(
      pl.kernel,
      mesh=mesh,
      out_type=[
          jax.ShapeDtypeStruct((_BATCH, 128), jnp.float32),
          jax.ShapeDtypeStruct((_BATCH, 128), jnp.float32),
      ],
      scratch_types=[
          pltpu.VMEM((_PER_WORKER,), jnp.int32),
          pltpu.VMEM((_PER_WORKER,), jnp.int32),
          pltpu.VMEM((_PER_WORKER,), jnp.int32),
          pltpu.VMEM((_PER_WORKER,), jnp.int32),
          pltpu.VMEM((_PER_WORKER, 128), jnp.float32),
          pltpu.VMEM((_PER_WORKER, 128), jnp.float32),
          pltpu.SemaphoreType.DMA,
          pltpu.SemaphoreType.DMA,
      ],
  )
  def gather_kernel(tgt_hbm, ctx_hbm, table_hbm, out_t, out_c,
                    idx_t, idx_c, gid_t, gid_c, rows_t, rows_c,
                    sem_t, sem_c):
    wid = lax.axis_index("s") * _NUM_CORES + lax.axis_index("c")
    base = wid * _PER_WORKER
    sl = pl.ds(base, _PER_WORKER)
    pltpu.sync_copy(tgt_hbm.at[sl], idx_t)
    pltpu.sync_copy(ctx_hbm.at[sl], idx_c)
    for k in range(_PER_WORKER // _LANES):
      ksl = pl.ds(k * _LANES, _LANES)
      gid_t[ksl] = lax.shift_right_logical(idx_t[ksl], 2)
      gid_c[ksl] = lax.shift_right_logical(idx_c[ksl], 2)
    cp_t = pltpu.async_copy(table_hbm.at[gid_t], rows_t, sem_t)
    cp_c = pltpu.async_copy(table_hbm.at[gid_c], rows_c, sem_c)
    cp_t.wait()
    pltpu.sync_copy(rows_t, out_t.at[sl])
    cp_c.wait()
    pltpu.sync_copy(rows_c, out_c.at[sl])

  return gather_kernel


_gather = _make_gather()

_BSEL = 512


def _select_body(raw_ref, off_ref, o_ref):
  raw = raw_ref[...]
  off = off_ref[...]
  acc = jnp.zeros((_BSEL, _EMBED), jnp.float32)
  for c in range(_GROUP):
    acc = acc + jnp.where(off == c, raw[:, c * _EMBED:(c + 1) * _EMBED], 0.0)
  o_ref[...] = acc


_select = pl.pallas_call(
    _select_body,
    grid=(_BATCH // _BSEL,),
    in_specs=[
        pl.BlockSpec((_BSEL, 128), lambda i: (i, 0)),
        pl.BlockSpec((_BSEL, 1), lambda i: (i, 0)),
    ],
    out_specs=pl.BlockSpec((_BSEL, _EMBED), lambda i: (i, 0)),
    out_shape=jax.ShapeDtypeStruct((_BATCH, _EMBED), jnp.float32),
)

_BM = 256


def _matmul_body(a_ref, b_ref, o_ref):
  o_ref[...] = lax.dot_general(
      a_ref[...], b_ref[...],
      dimension_numbers=(((1,), (1,)), ((), ())),
      preferred_element_type=jnp.float32)


_matmul = pl.pallas_call(
    _matmul_body,
    grid=(_BATCH // _BM,),
    in_specs=[
        pl.BlockSpec((_BM, _EMBED), lambda i: (i, 0)),
        pl.BlockSpec((_BATCH, _EMBED), lambda i: (0, 0)),
    ],
    out_specs=pl.BlockSpec((_BM, _BATCH), lambda i: (i, 0)),
    out_shape=jax.ShapeDtypeStruct((_BATCH, _BATCH), jnp.float32),
)


@jax.jit
def kernel(target, context, embeddings):
  target = target.astype(jnp.int32)
  context = context.astype(jnp.int32)
  table_g = embeddings.reshape(_NGROUPS, 128)
  raw_t, raw_c = _gather(target, context, table_g)
  off_t = (target & (_GROUP - 1)).reshape(_BATCH, 1)
  off_c = (context & (_GROUP - 1)).reshape(_BATCH, 1)
  rows_t = _select(raw_t, off_t)
  rows_c = _select(raw_c, off_c)
  return _matmul(rows_t, rows_c)


# speedup vs baseline: 8.0229x; 8.0229x over previous
import functools

import jax
import jax.numpy as jnp
from jax import lax
from jax.experimental import pallas as pl
from jax.experimental.pallas import tpu as pltpu
from jax.experimental.pallas import tpu_sc as plsc

_VOCAB = 1000000
_EMBED = 32
_BATCH = 4096
_GROUP = 128 // _EMBED
_NGROUPS = _VOCAB // _GROUP

_NUM_CORES = 2
_NUM_SUBCORES = 16
_NUM_WORKERS = _NUM_CORES * _NUM_SUBCORES
_PER_WORKER = _BATCH // _NUM_WORKERS
_LANES = 16


def _make_gather():
  mesh = plsc.VectorSubcoreMesh(
      core_axis_name="c", subcore_axis_name="s",
      num_cores=_NUM_CORES, num_subcores=_NUM_SUBCORES)

  @functools.partial(
      pl.kernel,
      mesh=mesh,
      out_type=[
          jax.ShapeDtypeStruct((_BATCH, 128), jnp.float32),
          jax.ShapeDtypeStruct((_BATCH, 128), jnp.float32),
      ],
      scratch_types=[
          pltpu.VMEM((_PER_WORKER,), jnp.int32),
          pltpu.VMEM((_PER_WORKER,), jnp.int32),
          pltpu.VMEM((_PER_WORKER,), jnp.int32),
          pltpu.VMEM((_PER_WORKER,), jnp.int32),
          pltpu.VMEM((_PER_WORKER, 128), jnp.float32),
          pltpu.VMEM((_PER_WORKER, 128), jnp.float32),
          pltpu.SemaphoreType.DMA,
          pltpu.SemaphoreType.DMA,
      ],
  )
  def gather_kernel(tgt_hbm, ctx_hbm, table_hbm, out_t, out_c,
                    idx_t, idx_c, gid_t, gid_c, rows_t, rows_c,
                    sem_t, sem_c):
    wid = lax.axis_index("s") * _NUM_CORES + lax.axis_index("c")
    base = wid * _PER_WORKER
    sl = pl.ds(base, _PER_WORKER)
    pltpu.sync_copy(tgt_hbm.at[sl], idx_t)
    pltpu.sync_copy(ctx_hbm.at[sl], idx_c)
    for k in range(_PER_WORKER // _LANES):
      ksl = pl.ds(k * _LANES, _LANES)
      gid_t[ksl] = lax.shift_right_logical(idx_t[ksl], 2)
      gid_c[ksl] = lax.shift_right_logical(idx_c[ksl], 2)
    cp_t = pltpu.async_copy(table_hbm.at[gid_t], rows_t, sem_t)
    cp_c = pltpu.async_copy(table_hbm.at[gid_c], rows_c, sem_c)
    cp_t.wait()
    pltpu.sync_copy(rows_t, out_t.at[sl])
    cp_c.wait()
    pltpu.sync_copy(rows_c, out_c.at[sl])

  return gather_kernel


_gather = _make_gather()

_BSEL = 512


def _select_body(raw_ref, off_ref, o_ref):
  raw = raw_ref[...]
  off = off_ref[...]
  acc = jnp.zeros((_BSEL, _EMBED), jnp.float32)
  for c in range(_GROUP):
    acc = acc + jnp.where(off == c, raw[:, c * _EMBED:(c + 1) * _EMBED], 0.0)
  o_ref[...] = acc


_select = pl.pallas_call(
    _select_body,
    grid=(_BATCH // _BSEL,),
    in_specs=[
        pl.BlockSpec((_BSEL, 128), lambda i: (i, 0)),
        pl.BlockSpec((_BSEL, 1), lambda i: (i, 0)),
    ],
    out_specs=pl.BlockSpec((_BSEL, _EMBED), lambda i: (i, 0)),
    out_shape=jax.ShapeDtypeStruct((_BATCH, _EMBED), jnp.float32),
)

_BM = 256


def _matmul_body(a_ref, b_ref, o_ref):
  o_ref[...] = lax.dot_general(
      a_ref[...], b_ref[...],
      dimension_numbers=(((1,), (1,)), ((), ())),
      preferred_element_type=jnp.float32)


_matmul = pl.pallas_call(
    _matmul_body,
    grid=(_BATCH // _BM,),
    in_specs=[
        pl.BlockSpec((_BM, _EMBED), lambda i: (i, 0)),
        pl.BlockSpec((_BATCH, _EMBED), lambda i: (0, 0)),
    ],
    out_specs=pl.BlockSpec((_BM, _BATCH), lambda i: (i, 0)),
    out_shape=jax.ShapeDtypeStruct((_BATCH, _BATCH), jnp.float32),
)


@jax.jit
def kernel(target, context, embeddings):
  rows_t = jnp.take(embeddings, target, axis=0)
  rows_c = jnp.take(embeddings, context, axis=0)
  return _matmul(rows_t, rows_c)
